# baseline (speedup 1.0000x reference)
import functools

import jax
import jax.numpy as jnp
from jax import lax
from jax.experimental import pallas as pl
from jax.experimental.pallas import tpu as pltpu
from jax.experimental.pallas import tpu_sc as plsc

B = 16384
DIM = 32
PACK = 4
LINE = PACK * DIM


def _sc_gather4(qu, qi, p_all):
    info = plsc.get_sparse_core_info()
    NC, NS = info.num_cores, info.num_subcores
    NW = NC * NS
    b_per_w = B // NW
    mesh = plsc.VectorSubcoreMesh(core_axis_name="c", subcore_axis_name="s")

    @functools.partial(
        pl.kernel,
        mesh=mesh,
        compiler_params=pltpu.CompilerParams(skip_device_barrier=True),
        out_type=[jax.ShapeDtypeStruct((B, LINE), jnp.float32)] * 4,
        scratch_types=[
            pltpu.VMEM((b_per_w,), jnp.int32),
            pltpu.VMEM((b_per_w,), jnp.int32),
            pltpu.VMEM((b_per_w, LINE), jnp.float32),
            pltpu.SemaphoreType.DMA,
        ],
    )
    def gather_k(qu_h, qi_h, t_all,
                 o_umf, o_imf, o_umlp, o_imlp, qu_v, qi_v, buf_v, sem):
        wid = lax.axis_index("s") * NC + lax.axis_index("c")
        base = wid * b_per_w
        sl = pl.ds(base, b_per_w)
        pltpu.sync_copy(qu_h.at[sl], qu_v)
        pltpu.sync_copy(qi_h.at[sl], qi_v)

        def one_table(q_v, t, out_h):
            tbl_h = t_all.at[t]
            for k in range(b_per_w // 128):
                pltpu.async_copy(
                    tbl_h.at[q_v.at[pl.ds(k * 128, 128)]],
                    buf_v.at[pl.ds(k * 128, 128)],
                    sem,
                )
            pltpu.make_async_copy(tbl_h.at[pl.ds(0, b_per_w)], buf_v,
                                  sem).wait()
            pltpu.sync_copy(buf_v, out_h.at[sl])

        one_table(qu_v, 0, o_umf)
        one_table(qi_v, 1, o_imf)
        one_table(qu_v, 2, o_umlp)
        one_table(qi_v, 3, o_imlp)

    return gather_k(qu, qi, p_all)


def _tc_dense(g_umf, g_imf, g_umlp, g_imlp, fu, fi, W1a, W1b, b1, W2, b2,
              W3, b3, W4, b4, Wp_mf, Wp_h, bp):
    def unpack(x_r, onehot):
        acc = x_r[:, 0:DIM] * onehot[:, 0:1]
        for c in range(1, PACK):
            acc += x_r[:, c * DIM:(c + 1) * DIM] * onehot[:, c:c + 1]
        return acc

    def body(g_umf_r, g_imf_r, g_umlp_r, g_imlp_r, fu_r, fi_r, w1a_r,
             w1b_r, b1_r, w2_r, b2_r, w3_r, b3_r, w4_r, b4_r, wpm_r,
             wph_r, bp_r, o_r):
        f32 = jnp.float32
        cids = lax.broadcasted_iota(jnp.int32, (1, PACK), 1)
        oh_u = (fu_r[...] == cids).astype(f32)
        oh_i = (fi_r[...] == cids).astype(f32)
        u_mlp = unpack(g_umlp_r[...], oh_u)
        i_mlp = unpack(g_imlp_r[...], oh_i)
        h = jax.nn.relu(
            jnp.dot(u_mlp, w1a_r[...], preferred_element_type=f32)
            + jnp.dot(i_mlp, w1b_r[...], preferred_element_type=f32)
            + b1_r[...])
        h = jax.nn.relu(
            jnp.dot(h, w2_r[...], preferred_element_type=f32) + b2_r[...])
        h = jax.nn.relu(
            jnp.dot(h, w3_r[...], preferred_element_type=f32) + b3_r[...])
        h = jax.nn.relu(
            jnp.dot(h, w4_r[...], preferred_element_type=f32) + b4_r[...])
        mf = unpack(g_umf_r[...], oh_u) * unpack(g_imf_r[...], oh_i)
        o_r[...] = (
            jnp.dot(mf, wpm_r[...], preferred_element_type=f32)
            + jnp.dot(h, wph_r[...], preferred_element_type=f32)
            + bp_r[...])

    nblk = 8
    bb = B // nblk
    row_spec = pl.BlockSpec((bb, LINE), lambda i: (i, 0))
    one_spec = pl.BlockSpec((bb, 1), lambda i: (i, 0))
    full = lambda *shape: pl.BlockSpec(shape, lambda i: (0,) * len(shape))
    return pl.pallas_call(
        body,
        grid=(nblk,),
        in_specs=[row_spec, row_spec, row_spec, row_spec, one_spec, one_spec,
                  full(DIM, DIM), full(DIM, DIM), full(1, DIM),
                  full(DIM, DIM), full(1, DIM),
                  full(DIM, 16), full(1, 16),
                  full(16, 8), full(1, 8),
                  full(DIM, 1), full(8, 1), full(1, 1)],
        out_specs=one_spec,
        out_shape=jax.ShapeDtypeStruct((B, 1), jnp.float32),
    )(g_umf, g_imf, g_umlp, g_imlp, fu, fi, W1a, W1b, b1, W2, b2, W3, b3,
      W4, b4, Wp_mf, Wp_h, bp)


def kernel(user, item, ue_mf, ie_mf, ue_mlp, ie_mlp, W1, b1, W2, b2, W3, b3,
           W4, b4, Wp, bp):
    user = user.astype(jnp.int32)
    item = item.astype(jnp.int32)
    qu, fu = user >> 2, (user & 3).reshape(B, 1)
    qi, fi = item >> 2, (item & 3).reshape(B, 1)
    p_shape = (ue_mf.shape[0] // PACK, LINE)
    p_all = jnp.stack([ue_mf.reshape(p_shape), ie_mf.reshape(p_shape),
                       ue_mlp.reshape(p_shape), ie_mlp.reshape(p_shape)])
    g_umf, g_imf, g_umlp, g_imlp = _sc_gather4(qu, qi, p_all)
    W1a, W1b = W1[:DIM], W1[DIM:]
    Wp_mf, Wp_h = Wp[:DIM], Wp[DIM:]
    return _tc_dense(g_umf, g_imf, g_umlp, g_imlp, fu, fi, W1a, W1b,
                     b1.reshape(1, -1), W2, b2.reshape(1, -1),
                     W3, b3.reshape(1, -1), W4, b4.reshape(1, -1),
                     Wp_mf, Wp_h, bp.reshape(1, 1))


# speedup vs baseline: 1.1849x; 1.1849x over previous
import functools

import jax
import jax.numpy as jnp
from jax import lax
from jax.experimental import pallas as pl
from jax.experimental.pallas import tpu as pltpu
from jax.experimental.pallas import tpu_sc as plsc

B = 16384
DIM = 32
PACK = 4
LINE = PACK * DIM


def _sc_gather4(qu, qi, p_umf, p_imf, p_umlp, p_imlp):
    info = plsc.get_sparse_core_info()
    NC, NS = info.num_cores, info.num_subcores
    NW = NC * NS
    b_per_w = B // NW
    mesh = plsc.VectorSubcoreMesh(core_axis_name="c", subcore_axis_name="s")

    @functools.partial(
        pl.kernel,
        mesh=mesh,
        out_type=[jax.ShapeDtypeStruct((B, LINE), jnp.float32)] * 4,
        scratch_types=[
            pltpu.VMEM((b_per_w,), jnp.int32),
            pltpu.VMEM((b_per_w,), jnp.int32),
            pltpu.VMEM((b_per_w, LINE), jnp.float32),
            pltpu.SemaphoreType.DMA,
        ],
    )
    def gather_k(qu_h, qi_h, t_umf, t_imf, t_umlp, t_imlp,
                 o_umf, o_imf, o_umlp, o_imlp, qu_v, qi_v, buf_v, sem):
        wid = lax.axis_index("s") * NC + lax.axis_index("c")
        base = wid * b_per_w
        sl = pl.ds(base, b_per_w)
        pltpu.sync_copy(qu_h.at[sl], qu_v)
        pltpu.sync_copy(qi_h.at[sl], qi_v)

        def one_table(q_v, tbl_h, out_h):
            for k in range(b_per_w // 128):
                pltpu.async_copy(
                    tbl_h.at[q_v.at[pl.ds(k * 128, 128)]],
                    buf_v.at[pl.ds(k * 128, 128)],
                    sem,
                )
            pltpu.make_async_copy(tbl_h.at[pl.ds(0, b_per_w)], buf_v,
                                  sem).wait()
            pltpu.sync_copy(buf_v, out_h.at[sl])

        one_table(qu_v, t_umf, o_umf)
        one_table(qi_v, t_imf, o_imf)
        one_table(qu_v, t_umlp, o_umlp)
        one_table(qi_v, t_imlp, o_imlp)

    return gather_k(qu, qi, p_umf, p_imf, p_umlp, p_imlp)


def _tc_dense(g_umf, g_imf, g_umlp, g_imlp, fu, fi, W1a, W1b, b1, W2, b2,
              W3, b3, W4, b4, Wp_mf, Wp_h, bp):
    def unpack(x_r, onehot):
        acc = x_r[:, 0:DIM] * onehot[:, 0:1]
        for c in range(1, PACK):
            acc += x_r[:, c * DIM:(c + 1) * DIM] * onehot[:, c:c + 1]
        return acc

    def body(g_umf_r, g_imf_r, g_umlp_r, g_imlp_r, fu_r, fi_r, w1a_r,
             w1b_r, b1_r, w2_r, b2_r, w3_r, b3_r, w4_r, b4_r, wpm_r,
             wph_r, bp_r, o_r):
        f32 = jnp.float32
        cids = lax.broadcasted_iota(jnp.int32, (1, PACK), 1)
        oh_u = (fu_r[...] == cids).astype(f32)
        oh_i = (fi_r[...] == cids).astype(f32)
        u_mlp = unpack(g_umlp_r[...], oh_u)
        i_mlp = unpack(g_imlp_r[...], oh_i)
        h = jax.nn.relu(
            jnp.dot(u_mlp, w1a_r[...], preferred_element_type=f32)
            + jnp.dot(i_mlp, w1b_r[...], preferred_element_type=f32)
            + b1_r[...])
        h = jax.nn.relu(
            jnp.dot(h, w2_r[...], preferred_element_type=f32) + b2_r[...])
        h = jax.nn.relu(
            jnp.dot(h, w3_r[...], preferred_element_type=f32) + b3_r[...])
        h = jax.nn.relu(
            jnp.dot(h, w4_r[...], preferred_element_type=f32) + b4_r[...])
        mf = unpack(g_umf_r[...], oh_u) * unpack(g_imf_r[...], oh_i)
        o_r[...] = (
            jnp.dot(mf, wpm_r[...], preferred_element_type=f32)
            + jnp.dot(h, wph_r[...], preferred_element_type=f32)
            + bp_r[...])

    nblk = 8
    bb = B // nblk
    row_spec = pl.BlockSpec((bb, LINE), lambda i: (i, 0))
    one_spec = pl.BlockSpec((bb, 1), lambda i: (i, 0))
    full = lambda *shape: pl.BlockSpec(shape, lambda i: (0,) * len(shape))
    return pl.pallas_call(
        body,
        grid=(nblk,),
        in_specs=[row_spec, row_spec, row_spec, row_spec, one_spec, one_spec,
                  full(DIM, DIM), full(DIM, DIM), full(1, DIM),
                  full(DIM, DIM), full(1, DIM),
                  full(DIM, 16), full(1, 16),
                  full(16, 8), full(1, 8),
                  full(DIM, 1), full(8, 1), full(1, 1)],
        out_specs=one_spec,
        out_shape=jax.ShapeDtypeStruct((B, 1), jnp.float32),
    )(g_umf, g_imf, g_umlp, g_imlp, fu, fi, W1a, W1b, b1, W2, b2, W3, b3,
      W4, b4, Wp_mf, Wp_h, bp)


def kernel(user, item, ue_mf, ie_mf, ue_mlp, ie_mlp, W1, b1, W2, b2, W3, b3,
           W4, b4, Wp, bp):
    user = user.astype(jnp.int32)
    item = item.astype(jnp.int32)
    qu, fu = user >> 2, (user & 3).reshape(B, 1)
    qi, fi = item >> 2, (item & 3).reshape(B, 1)
    p_shape = (ue_mf.shape[0] // PACK, LINE)
    g_umf, g_imf, g_umlp, g_imlp = _sc_gather4(
        qu, qi, ue_mf.reshape(p_shape), ie_mf.reshape(p_shape),
        ue_mlp.reshape(p_shape), ie_mlp.reshape(p_shape))
    W1a, W1b = W1[:DIM], W1[DIM:]
    Wp_mf, Wp_h = Wp[:DIM], Wp[DIM:]
    return _tc_dense(g_umf, g_imf, g_umlp, g_imlp, fu, fi, W1a, W1b,
                     b1.reshape(1, -1), W2, b2.reshape(1, -1),
                     W3, b3.reshape(1, -1), W4, b4.reshape(1, -1),
                     Wp_mf, Wp_h, bp.reshape(1, 1))


# speedup vs baseline: 1.1856x; 1.0006x over previous
import functools

import jax
import jax.numpy as jnp
from jax import lax
from jax.experimental import pallas as pl
from jax.experimental.pallas import tpu as pltpu
from jax.experimental.pallas import tpu_sc as plsc

B = 16384
DIM = 32
PACK = 4
LINE = PACK * DIM


def _sc_gather4(qu, qi, p_umf, p_imf, p_umlp, p_imlp):
    info = plsc.get_sparse_core_info()
    NC, NS = info.num_cores, info.num_subcores
    NW = NC * NS
    b_per_w = B // NW
    mesh = plsc.VectorSubcoreMesh(core_axis_name="c", subcore_axis_name="s")

    @functools.partial(
        pl.kernel,
        mesh=mesh,
        compiler_params=pltpu.CompilerParams(skip_device_barrier=True),
        out_type=[jax.ShapeDtypeStruct((B, LINE), jnp.float32)] * 4,
        scratch_types=[
            pltpu.VMEM((b_per_w,), jnp.int32),
            pltpu.VMEM((b_per_w,), jnp.int32),
            pltpu.VMEM((b_per_w, LINE), jnp.float32),
            pltpu.SemaphoreType.DMA,
        ],
    )
    def gather_k(qu_h, qi_h, t_umf, t_imf, t_umlp, t_imlp,
                 o_umf, o_imf, o_umlp, o_imlp, qu_v, qi_v, buf_v, sem):
        wid = lax.axis_index("s") * NC + lax.axis_index("c")
        base = wid * b_per_w
        sl = pl.ds(base, b_per_w)
        pltpu.sync_copy(qu_h.at[sl], qu_v)
        pltpu.sync_copy(qi_h.at[sl], qi_v)

        def one_table(q_v, tbl_h, out_h):
            for k in range(b_per_w // 128):
                pltpu.async_copy(
                    tbl_h.at[q_v.at[pl.ds(k * 128, 128)]],
                    buf_v.at[pl.ds(k * 128, 128)],
                    sem,
                )
            pltpu.make_async_copy(tbl_h.at[pl.ds(0, b_per_w)], buf_v,
                                  sem).wait()
            pltpu.sync_copy(buf_v, out_h.at[sl])

        one_table(qu_v, t_umf, o_umf)
        one_table(qi_v, t_imf, o_imf)
        one_table(qu_v, t_umlp, o_umlp)
        one_table(qi_v, t_imlp, o_imlp)

    return gather_k(qu, qi, p_umf, p_imf, p_umlp, p_imlp)


def _tc_dense(g_umf, g_imf, g_umlp, g_imlp, fu, fi, W1a, W1b, b1, W2, b2,
              W3, b3, W4, b4, Wp_mf, Wp_h, bp):
    def unpack(x_r, onehot):
        acc = x_r[:, 0:DIM] * onehot[:, 0:1]
        for c in range(1, PACK):
            acc += x_r[:, c * DIM:(c + 1) * DIM] * onehot[:, c:c + 1]
        return acc

    def body(g_umf_r, g_imf_r, g_umlp_r, g_imlp_r, fu_r, fi_r, w1a_r,
             w1b_r, b1_r, w2_r, b2_r, w3_r, b3_r, w4_r, b4_r, wpm_r,
             wph_r, bp_r, o_r):
        f32 = jnp.float32
        cids = lax.broadcasted_iota(jnp.int32, (1, PACK), 1)
        oh_u = (fu_r[...] == cids).astype(f32)
        oh_i = (fi_r[...] == cids).astype(f32)
        u_mlp = unpack(g_umlp_r[...], oh_u)
        i_mlp = unpack(g_imlp_r[...], oh_i)
        h = jax.nn.relu(
            jnp.dot(u_mlp, w1a_r[...], preferred_element_type=f32)
            + jnp.dot(i_mlp, w1b_r[...], preferred_element_type=f32)
            + b1_r[...])
        h = jax.nn.relu(
            jnp.dot(h, w2_r[...], preferred_element_type=f32) + b2_r[...])
        h = jax.nn.relu(
            jnp.dot(h, w3_r[...], preferred_element_type=f32) + b3_r[...])
        h = jax.nn.relu(
            jnp.dot(h, w4_r[...], preferred_element_type=f32) + b4_r[...])
        mf = unpack(g_umf_r[...], oh_u) * unpack(g_imf_r[...], oh_i)
        o_r[...] = (
            jnp.dot(mf, wpm_r[...], preferred_element_type=f32)
            + jnp.dot(h, wph_r[...], preferred_element_type=f32)
            + bp_r[...])

    nblk = 8
    bb = B // nblk
    row_spec = pl.BlockSpec((bb, LINE), lambda i: (i, 0))
    one_spec = pl.BlockSpec((bb, 1), lambda i: (i, 0))
    full = lambda *shape: pl.BlockSpec(shape, lambda i: (0,) * len(shape))
    return pl.pallas_call(
        body,
        grid=(nblk,),
        in_specs=[row_spec, row_spec, row_spec, row_spec, one_spec, one_spec,
                  full(DIM, DIM), full(DIM, DIM), full(1, DIM),
                  full(DIM, DIM), full(1, DIM),
                  full(DIM, 16), full(1, 16),
                  full(16, 8), full(1, 8),
                  full(DIM, 1), full(8, 1), full(1, 1)],
        out_specs=one_spec,
        out_shape=jax.ShapeDtypeStruct((B, 1), jnp.float32),
    )(g_umf, g_imf, g_umlp, g_imlp, fu, fi, W1a, W1b, b1, W2, b2, W3, b3,
      W4, b4, Wp_mf, Wp_h, bp)


def kernel(user, item, ue_mf, ie_mf, ue_mlp, ie_mlp, W1, b1, W2, b2, W3, b3,
           W4, b4, Wp, bp):
    user = user.astype(jnp.int32)
    item = item.astype(jnp.int32)
    qu, fu = user >> 2, (user & 3).reshape(B, 1)
    qi, fi = item >> 2, (item & 3).reshape(B, 1)
    p_shape = (ue_mf.shape[0] // PACK, LINE)
    g_umf, g_imf, g_umlp, g_imlp = _sc_gather4(
        qu, qi, ue_mf.reshape(p_shape), ie_mf.reshape(p_shape),
        ue_mlp.reshape(p_shape), ie_mlp.reshape(p_shape))
    W1a, W1b = W1[:DIM], W1[DIM:]
    Wp_mf, Wp_h = Wp[:DIM], Wp[DIM:]
    return _tc_dense(g_umf, g_imf, g_umlp, g_imlp, fu, fi, W1a, W1b,
                     b1.reshape(1, -1), W2, b2.reshape(1, -1),
                     W3, b3.reshape(1, -1), W4, b4.reshape(1, -1),
                     Wp_mf, Wp_h, bp.reshape(1, 1))
